# baseline (speedup 1.0000x reference)
import functools

import jax
import jax.numpy as jnp
from jax import lax
from jax.experimental import pallas as pl
from jax.experimental.pallas import tpu as pltpu
from jax.experimental.pallas import tpu_sc as plsc

_BINS = 50
_EPS = 1e-08

_NC = 2
_NS = 16
_L = 16

_BINS_PAD = 64
_NCOPY = 8
_HWORDS = _BINS_PAD * _L * _NCOPY
_SC_CHUNK = 32768


def _pass1_body(nchunk, rblk, p_ref, t_ref, s_ref, mm_ref):
    j = pl.program_id(0)
    vals = []
    for r in range(rblk):
        p0 = p_ref[:, 0, r, :]
        p1 = p_ref[:, 1, r, :]
        t0 = t_ref[:, 0, r, :]
        t1 = t_ref[:, 1, r, :]
        vals.append(p0 * p0 + p1 * p1)
        vals.append(t0 * t0 + t1 * t1)
    bmin = jnp.min(functools.reduce(jnp.minimum, vals))
    bmax = jnp.max(functools.reduce(jnp.maximum, vals))

    @pl.when(j == 0)
    def _():
        mm_ref[0] = bmin
        mm_ref[1] = bmax

    @pl.when(j > 0)
    def _():
        mm_ref[0] = jnp.minimum(mm_ref[0], bmin)
        mm_ref[1] = jnp.maximum(mm_ref[1], bmax)

    @pl.when(j == nchunk - 1)
    def _():
        mn = jnp.maximum(jnp.sqrt(mm_ref[0]), 1e-6)
        mx = jnp.maximum(jnp.sqrt(mm_ref[1]), 1e-6)
        scale = _BINS / jnp.maximum(mx - mn, 1e-12)
        s_ref[0, 0] = mn
        s_ref[0, 1] = scale


def _binpass_body(rblk, p_ref, t_ref, s_ref, o_ref):
    mn = s_ref[0, 0]
    sc = s_ref[0, 1]
    w = p_ref.shape[-1]
    ci = lax.broadcasted_iota(jnp.int32, (2 * 16, w), 1)
    pat = (ci & (_L - 1)) + ((ci >> 4) & (_NCOPY - 1)) * (_BINS_PAD * _L)
    for r in range(rblk):
        p0 = p_ref[:, 0, r, :]
        p1 = p_ref[:, 1, r, :]
        t0 = t_ref[:, 0, r, :]
        t1 = t_ref[:, 1, r, :]
        vp = jnp.maximum(jnp.sqrt(p0 * p0 + p1 * p1), 1e-6)
        vt = jnp.maximum(jnp.sqrt(t0 * t0 + t1 * t1), 1e-6)
        v = jnp.concatenate([vp, vt], axis=0)
        t = (v - mn) * sc
        t = jnp.maximum(jnp.minimum(t, float(_BINS - 1)), 0.0)
        idx = t.astype(jnp.int32)
        o_ref[:, r * w : (r + 1) * w] = idx * _L + pat


def _sc_hist_body(n, fidx, out, buf0, buf1, hist, sem0, sem1):
    c = lax.axis_index("c")
    s = lax.axis_index("s")
    wid = c * _NS + s

    zv = jnp.zeros((_L,), jnp.float32)

    @plsc.parallel_loop(0, _HWORDS // _L, unroll=4)
    def _zero(i):
        hist[pl.ds(i * _L, _L)] = zv

    ones = jnp.ones((_L,), jnp.float32)
    nchunk = n // _SC_CHUNK
    nvec = _SC_CHUNK // _L
    bufs = (buf0, buf1)
    sems = (sem0, sem1)
    copies = [None, None]
    copies[0] = pltpu.async_copy(fidx.at[wid, pl.ds(0, _SC_CHUNK)], buf0, sem0)
    for k in range(nchunk):
        cur = k % 2
        if k + 1 < nchunk:
            copies[1 - cur] = pltpu.async_copy(
                fidx.at[wid, pl.ds((k + 1) * _SC_CHUNK, _SC_CHUNK)],
                bufs[1 - cur],
                sems[1 - cur],
            )
        copies[cur].wait()
        buf = bufs[cur]

        @plsc.parallel_loop(0, nvec, unroll=8)
        def body(i):
            fl = buf[pl.ds(i * _L, _L)]
            plsc.addupdate_scatter(hist, [fl], ones)

    pltpu.sync_copy(hist, out.at[wid])


def _pass2_body(h_ref, o_ref):
    xs = h_ref[...]
    x = xs[:, 0 : _BINS_PAD * _L]
    for j in range(1, _NCOPY):
        x = x + xs[:, j * _BINS_PAD * _L : (j + 1) * _BINS_PAD * _L]
    rows = lax.broadcasted_iota(jnp.int32, (_BINS_PAD * _L, 128), 0)
    cols = lax.broadcasted_iota(jnp.int32, (_BINS_PAD * _L, 128), 1)
    m = (rows // _L == cols).astype(jnp.float32)
    h = jnp.dot(x, m, preferred_element_type=jnp.float32)
    lanes = lax.broadcasted_iota(jnp.int32, (16, 128), 1)
    valid = lanes < _BINS
    hp = jnp.where(valid, h[0:16, :] + _EPS, 0.0)
    ht = jnp.where(valid, h[16:32, :] + _EPS, 0.0)
    hp = hp / jnp.sum(hp, axis=1, keepdims=True)
    ht = ht / jnp.sum(ht, axis=1, keepdims=True)
    kl = jnp.where(valid, ht * jnp.log(ht / hp), 0.0)
    o_ref[0, 0] = jnp.sum(kl) / 16.0


def kernel(y_pred, y_true):
    B, C, H, W = y_pred.shape
    N = H * W

    ch = 4096
    rblk = ch // W
    nchunk = N // ch
    stats = pl.pallas_call(
        functools.partial(_pass1_body, nchunk, rblk),
        grid=(nchunk,),
        in_specs=[
            pl.BlockSpec((B, C, rblk, W), lambda j: (0, 0, j, 0)),
            pl.BlockSpec((B, C, rblk, W), lambda j: (0, 0, j, 0)),
        ],
        out_specs=pl.BlockSpec((1, 2), lambda j: (0, 0), memory_space=pltpu.SMEM),
        out_shape=jax.ShapeDtypeStruct((1, 2), jnp.float32),
        scratch_shapes=[pltpu.SMEM((2,), jnp.float32)],
    )(y_pred, y_true)

    fidx = pl.pallas_call(
        functools.partial(_binpass_body, rblk),
        grid=(nchunk,),
        in_specs=[
            pl.BlockSpec((B, C, rblk, W), lambda j: (0, 0, j, 0)),
            pl.BlockSpec((B, C, rblk, W), lambda j: (0, 0, j, 0)),
            pl.BlockSpec((1, 2), lambda j: (0, 0), memory_space=pltpu.SMEM),
        ],
        out_specs=pl.BlockSpec((2 * B, ch), lambda j: (0, j)),
        out_shape=jax.ShapeDtypeStruct((2 * B, N), jnp.int32),
    )(y_pred, y_true, stats)

    mesh = plsc.VectorSubcoreMesh(
        core_axis_name="c", subcore_axis_name="s", num_cores=_NC, num_subcores=_NS
    )
    sc_hist = functools.partial(
        pl.kernel,
        out_type=jax.ShapeDtypeStruct((2 * B, _HWORDS), jnp.float32),
        mesh=mesh,
        compiler_params=pltpu.CompilerParams(needs_layout_passes=False),
        scratch_types=[
            pltpu.VMEM((_SC_CHUNK,), jnp.int32),
            pltpu.VMEM((_SC_CHUNK,), jnp.int32),
            pltpu.VMEM((_HWORDS,), jnp.float32),
            pltpu.SemaphoreType.DMA,
            pltpu.SemaphoreType.DMA,
        ],
    )(functools.partial(_sc_hist_body, N))
    hists = sc_hist(fidx)

    out = pl.pallas_call(
        _pass2_body,
        in_specs=[pl.BlockSpec((2 * B, _HWORDS), lambda: (0, 0))],
        out_specs=pl.BlockSpec(memory_space=pltpu.SMEM),
        out_shape=jax.ShapeDtypeStruct((1, 1), jnp.float32),
    )(hists)
    return out[0, 0]


# speedup vs baseline: 1.1217x; 1.1217x over previous
import functools

import jax
import jax.numpy as jnp
from jax import lax
from jax.experimental import pallas as pl
from jax.experimental.pallas import tpu as pltpu
from jax.experimental.pallas import tpu_sc as plsc

_BINS = 50
_EPS = 1e-08

_NC = 2
_NS = 16
_L = 16

_BINS_PAD = 64
_NCOPY = 8
_HWORDS = _BINS_PAD * _L * _NCOPY
_SC_CHUNK = 32768


def _pass1_body(nchunk, rblk, p_ref, t_ref, s_ref, mm_ref):
    j = pl.program_id(0)
    vals = []
    for r in range(rblk):
        p0 = p_ref[:, 0, r, :]
        p1 = p_ref[:, 1, r, :]
        t0 = t_ref[:, 0, r, :]
        t1 = t_ref[:, 1, r, :]
        vals.append(p0 * p0 + p1 * p1)
        vals.append(t0 * t0 + t1 * t1)
    bmin = jnp.min(functools.reduce(jnp.minimum, vals))
    bmax = jnp.max(functools.reduce(jnp.maximum, vals))

    @pl.when(j == 0)
    def _():
        mm_ref[0] = bmin
        mm_ref[1] = bmax

    @pl.when(j > 0)
    def _():
        mm_ref[0] = jnp.minimum(mm_ref[0], bmin)
        mm_ref[1] = jnp.maximum(mm_ref[1], bmax)

    @pl.when(j == nchunk - 1)
    def _():
        mn = jnp.maximum(jnp.sqrt(mm_ref[0]), 1e-6)
        mx = jnp.maximum(jnp.sqrt(mm_ref[1]), 1e-6)
        scale = _BINS / jnp.maximum(mx - mn, 1e-12)
        s_ref[0, 0] = mn
        s_ref[0, 1] = scale


def _binpass_body(rblk, p_ref, s_ref, o_ref):
    mn = s_ref[0, 0]
    sc = s_ref[0, 1]
    w = p_ref.shape[-1]
    ci = lax.broadcasted_iota(jnp.int32, (16, w), 1)
    pat = (ci & (_L - 1)) + ((ci >> 4) & (_NCOPY - 1)) * (_BINS_PAD * _L)
    for r in range(rblk):
        p0 = p_ref[:, 0, r, :]
        p1 = p_ref[:, 1, r, :]
        v = jnp.maximum(jnp.sqrt(p0 * p0 + p1 * p1), 1e-6)
        t = (v - mn) * sc
        t = jnp.maximum(jnp.minimum(t, float(_BINS - 1)), 0.0)
        idx = t.astype(jnp.int32)
        o_ref[:, r * w : (r + 1) * w] = idx * _L + pat


def _sc_hist_body(n, fidx, out, buf0, buf1, hist, sem0, sem1):
    c = lax.axis_index("c")
    s = lax.axis_index("s")
    wid = c * _NS + s
    half = n // 2
    base = c * half

    zv = jnp.zeros((_L,), jnp.float32)

    @plsc.parallel_loop(0, _HWORDS // _L, unroll=4)
    def _zero(i):
        hist[pl.ds(i * _L, _L)] = zv

    ones = jnp.ones((_L,), jnp.float32)
    nchunk = half // _SC_CHUNK
    nvec = _SC_CHUNK // _L
    bufs = (buf0, buf1)
    sems = (sem0, sem1)
    copies = [None, None]
    copies[0] = pltpu.async_copy(fidx.at[s, pl.ds(base, _SC_CHUNK)], buf0, sem0)
    for k in range(nchunk):
        cur = k % 2
        if k + 1 < nchunk:
            copies[1 - cur] = pltpu.async_copy(
                fidx.at[s, pl.ds(base + (k + 1) * _SC_CHUNK, _SC_CHUNK)],
                bufs[1 - cur],
                sems[1 - cur],
            )
        copies[cur].wait()
        buf = bufs[cur]

        @plsc.parallel_loop(0, nvec, unroll=8)
        def body(i):
            fl = buf[pl.ds(i * _L, _L)]
            plsc.addupdate_scatter(hist, [fl], ones)

    pltpu.sync_copy(hist, out.at[wid])


def _fold_partials(xs):
    x = xs[0:16, :] + xs[16:32, :]
    acc = x[:, 0 : _BINS_PAD * _L]
    for j in range(1, _NCOPY):
        acc = acc + x[:, j * _BINS_PAD * _L : (j + 1) * _BINS_PAD * _L]
    return acc


def _pass2_body(hp_ref, ht_ref, o_ref):
    x = jnp.concatenate([_fold_partials(hp_ref[...]), _fold_partials(ht_ref[...])], axis=0)
    rows = lax.broadcasted_iota(jnp.int32, (_BINS_PAD * _L, 128), 0)
    cols = lax.broadcasted_iota(jnp.int32, (_BINS_PAD * _L, 128), 1)
    m = (rows // _L == cols).astype(jnp.float32)
    h = jnp.dot(x, m, preferred_element_type=jnp.float32)
    lanes = lax.broadcasted_iota(jnp.int32, (16, 128), 1)
    valid = lanes < _BINS
    hp = jnp.where(valid, h[0:16, :] + _EPS, 0.0)
    ht = jnp.where(valid, h[16:32, :] + _EPS, 0.0)
    hp = hp / jnp.sum(hp, axis=1, keepdims=True)
    ht = ht / jnp.sum(ht, axis=1, keepdims=True)
    kl = jnp.where(valid, ht * jnp.log(ht / hp), 0.0)
    o_ref[0, 0] = jnp.sum(kl) / 16.0


def kernel(y_pred, y_true):
    B, C, H, W = y_pred.shape
    N = H * W

    ch = 8192
    rblk = ch // W
    nchunk = N // ch
    stats = pl.pallas_call(
        functools.partial(_pass1_body, nchunk, rblk),
        grid=(nchunk,),
        in_specs=[
            pl.BlockSpec((B, C, rblk, W), lambda j: (0, 0, j, 0)),
            pl.BlockSpec((B, C, rblk, W), lambda j: (0, 0, j, 0)),
        ],
        out_specs=pl.BlockSpec((1, 2), lambda j: (0, 0), memory_space=pltpu.SMEM),
        out_shape=jax.ShapeDtypeStruct((1, 2), jnp.float32),
        scratch_shapes=[pltpu.SMEM((2,), jnp.float32)],
    )(y_pred, y_true)

    def binpass(arr):
        return pl.pallas_call(
            functools.partial(_binpass_body, rblk),
            grid=(nchunk,),
            in_specs=[
                pl.BlockSpec((B, C, rblk, W), lambda j: (0, 0, j, 0)),
                pl.BlockSpec((1, 2), lambda j: (0, 0), memory_space=pltpu.SMEM),
            ],
            out_specs=pl.BlockSpec((B, ch), lambda j: (0, j)),
            out_shape=jax.ShapeDtypeStruct((B, N), jnp.int32),
        )(arr, stats)

    mesh = plsc.VectorSubcoreMesh(
        core_axis_name="c", subcore_axis_name="s", num_cores=_NC, num_subcores=_NS
    )
    sc_hist = functools.partial(
        pl.kernel,
        out_type=jax.ShapeDtypeStruct((2 * B, _HWORDS), jnp.float32),
        mesh=mesh,
        compiler_params=pltpu.CompilerParams(needs_layout_passes=False),
        scratch_types=[
            pltpu.VMEM((_SC_CHUNK,), jnp.int32),
            pltpu.VMEM((_SC_CHUNK,), jnp.int32),
            pltpu.VMEM((_HWORDS,), jnp.float32),
            pltpu.SemaphoreType.DMA,
            pltpu.SemaphoreType.DMA,
        ],
    )(functools.partial(_sc_hist_body, N))

    hists_p = sc_hist(binpass(y_pred))
    hists_t = sc_hist(binpass(y_true))

    out = pl.pallas_call(
        _pass2_body,
        in_specs=[
            pl.BlockSpec((2 * B, _HWORDS), lambda: (0, 0)),
            pl.BlockSpec((2 * B, _HWORDS), lambda: (0, 0)),
        ],
        out_specs=pl.BlockSpec(memory_space=pltpu.SMEM),
        out_shape=jax.ShapeDtypeStruct((1, 1), jnp.float32),
    )(hists_p, hists_t)
    return out[0, 0]


# speedup vs baseline: 1.2046x; 1.0739x over previous
import functools

import jax
import jax.numpy as jnp
from jax import lax
from jax.experimental import pallas as pl
from jax.experimental.pallas import tpu as pltpu
from jax.experimental.pallas import tpu_sc as plsc

_BINS = 50
_EPS = 1e-08

_NC = 2
_NS = 16
_L = 16

_BINS_PAD = 64
_NCOPY = 8
_HWORDS = _BINS_PAD * _L * _NCOPY
_SC_CHUNK = 32768


def _pass1_body(nchunk, rblk, p_ref, t_ref, s_ref, mm_ref):
    j = pl.program_id(0)
    vals = []
    for r in range(rblk):
        p0 = p_ref[:, 0, r, :]
        p1 = p_ref[:, 1, r, :]
        t0 = t_ref[:, 0, r, :]
        t1 = t_ref[:, 1, r, :]
        vals.append(p0 * p0 + p1 * p1)
        vals.append(t0 * t0 + t1 * t1)
    bmin = jnp.min(functools.reduce(jnp.minimum, vals))
    bmax = jnp.max(functools.reduce(jnp.maximum, vals))

    @pl.when(j == 0)
    def _():
        mm_ref[0] = bmin
        mm_ref[1] = bmax

    @pl.when(j > 0)
    def _():
        mm_ref[0] = jnp.minimum(mm_ref[0], bmin)
        mm_ref[1] = jnp.maximum(mm_ref[1], bmax)

    @pl.when(j == nchunk - 1)
    def _():
        mn = jnp.maximum(jnp.sqrt(mm_ref[0]), 1e-6)
        mx = jnp.maximum(jnp.sqrt(mm_ref[1]), 1e-6)
        scale = _BINS / jnp.maximum(mx - mn, 1e-12)
        s_ref[0, 0] = mn
        s_ref[0, 1] = scale


def _binpass_body(rblk, p_ref, s_ref, o_ref):
    mn = s_ref[0, 0]
    sc = s_ref[0, 1]
    w = p_ref.shape[-1]
    ci = lax.broadcasted_iota(jnp.int32, (16, w), 1)
    pat = (ci & (_L - 1)) + ((ci >> 4) & (_NCOPY - 1)) * (_BINS_PAD * _L)
    for r in range(rblk):
        p0 = p_ref[:, 0, r, :]
        p1 = p_ref[:, 1, r, :]
        v = jnp.maximum(jnp.sqrt(p0 * p0 + p1 * p1), 1e-6)
        t = (v - mn) * sc
        t = jnp.maximum(jnp.minimum(t, float(_BINS - 1)), 0.0)
        idx = t.astype(jnp.int32)
        o_ref[:, r * w : (r + 1) * w] = idx * _L + pat


def _sc_hist_body(n, fidx, out, buf0, buf1, hist, sem0, sem1):
    c = lax.axis_index("c")
    s = lax.axis_index("s")
    wid = c * _NS + s
    half = n // 2
    base = c * half

    zv = jnp.zeros((_L,), jnp.float32)

    @plsc.parallel_loop(0, _HWORDS // _L, unroll=4)
    def _zero(i):
        hist[pl.ds(i * _L, _L)] = zv

    ones = jnp.ones((_L,), jnp.float32)
    nchunk = half // _SC_CHUNK
    nvec = _SC_CHUNK // _L
    bufs = (buf0, buf1)
    sems = (sem0, sem1)
    copies = [None, None]
    copies[0] = pltpu.async_copy(fidx.at[s, pl.ds(base, _SC_CHUNK)], buf0, sem0)
    for k in range(nchunk):
        cur = k % 2
        if k + 1 < nchunk:
            copies[1 - cur] = pltpu.async_copy(
                fidx.at[s, pl.ds(base + (k + 1) * _SC_CHUNK, _SC_CHUNK)],
                bufs[1 - cur],
                sems[1 - cur],
            )
        copies[cur].wait()
        buf = bufs[cur]

        @plsc.parallel_loop(0, nvec, unroll=8)
        def body(i):
            fl = buf[pl.ds(i * _L, _L)]
            plsc.addupdate_scatter(hist, [fl], ones)

    pltpu.sync_copy(hist, out.at[wid])


def _fold_partials(xs):
    x = xs[0:16, :] + xs[16:32, :]
    acc = x[:, 0 : _BINS_PAD * _L]
    for j in range(1, _NCOPY):
        acc = acc + x[:, j * _BINS_PAD * _L : (j + 1) * _BINS_PAD * _L]
    return acc


def _pass2_body(hp_ref, ht1_ref, ht2_ref, o_ref):
    ht = _fold_partials(ht1_ref[...]) + _fold_partials(ht2_ref[...])
    x = jnp.concatenate([_fold_partials(hp_ref[...]), ht], axis=0)
    rows = lax.broadcasted_iota(jnp.int32, (_BINS_PAD * _L, 128), 0)
    cols = lax.broadcasted_iota(jnp.int32, (_BINS_PAD * _L, 128), 1)
    m = (rows // _L == cols).astype(jnp.float32)
    h = jnp.dot(x, m, preferred_element_type=jnp.float32)
    lanes = lax.broadcasted_iota(jnp.int32, (16, 128), 1)
    valid = lanes < _BINS
    hp = jnp.where(valid, h[0:16, :] + _EPS, 0.0)
    ht = jnp.where(valid, h[16:32, :] + _EPS, 0.0)
    hp = hp / jnp.sum(hp, axis=1, keepdims=True)
    ht = ht / jnp.sum(ht, axis=1, keepdims=True)
    kl = jnp.where(valid, ht * jnp.log(ht / hp), 0.0)
    o_ref[0, 0] = jnp.sum(kl) / 16.0


def kernel(y_pred, y_true):
    B, C, H, W = y_pred.shape
    N = H * W

    ch1 = 16384
    rblk1 = ch1 // W
    stats = pl.pallas_call(
        functools.partial(_pass1_body, N // ch1, rblk1),
        grid=(N // ch1,),
        in_specs=[
            pl.BlockSpec((B, C, rblk1, W), lambda j: (0, 0, j, 0)),
            pl.BlockSpec((B, C, rblk1, W), lambda j: (0, 0, j, 0)),
        ],
        out_specs=pl.BlockSpec((1, 2), lambda j: (0, 0), memory_space=pltpu.SMEM),
        out_shape=jax.ShapeDtypeStruct((1, 2), jnp.float32),
        scratch_shapes=[pltpu.SMEM((2,), jnp.float32)],
    )(y_pred, y_true)

    ch = 8192
    rblk = ch // W
    nchunk = N // ch

    def binpass(arr, r0, nr):
        return pl.pallas_call(
            functools.partial(_binpass_body, rblk),
            grid=(nr,),
            in_specs=[
                pl.BlockSpec((B, C, rblk, W), lambda j: (0, 0, r0 + j, 0)),
                pl.BlockSpec((1, 2), lambda j: (0, 0), memory_space=pltpu.SMEM),
            ],
            out_specs=pl.BlockSpec((B, ch), lambda j: (0, j)),
            out_shape=jax.ShapeDtypeStruct((B, nr * ch), jnp.int32),
        )(arr, stats)

    mesh = plsc.VectorSubcoreMesh(
        core_axis_name="c", subcore_axis_name="s", num_cores=_NC, num_subcores=_NS
    )

    def sc_hist(n):
        return functools.partial(
            pl.kernel,
            out_type=jax.ShapeDtypeStruct((2 * B, _HWORDS), jnp.float32),
            mesh=mesh,
            compiler_params=pltpu.CompilerParams(needs_layout_passes=False),
            scratch_types=[
                pltpu.VMEM((_SC_CHUNK,), jnp.int32),
                pltpu.VMEM((_SC_CHUNK,), jnp.int32),
                pltpu.VMEM((_HWORDS,), jnp.float32),
                pltpu.SemaphoreType.DMA,
                pltpu.SemaphoreType.DMA,
            ],
        )(functools.partial(_sc_hist_body, n))

    hists_p = sc_hist(N)(binpass(y_pred, 0, nchunk))
    hists_t1 = sc_hist(N // 2)(binpass(y_true, 0, nchunk // 2))
    hists_t2 = sc_hist(N // 2)(binpass(y_true, nchunk // 2, nchunk // 2))

    out = pl.pallas_call(
        _pass2_body,
        in_specs=[
            pl.BlockSpec((2 * B, _HWORDS), lambda: (0, 0)),
            pl.BlockSpec((2 * B, _HWORDS), lambda: (0, 0)),
            pl.BlockSpec((2 * B, _HWORDS), lambda: (0, 0)),
        ],
        out_specs=pl.BlockSpec(memory_space=pltpu.SMEM),
        out_shape=jax.ShapeDtypeStruct((1, 1), jnp.float32),
    )(hists_p, hists_t1, hists_t2)
    return out[0, 0]


# speedup vs baseline: 1.2461x; 1.0345x over previous
import functools

import jax
import jax.numpy as jnp
from jax import lax
from jax.experimental import pallas as pl
from jax.experimental.pallas import tpu as pltpu
from jax.experimental.pallas import tpu_sc as plsc

_BINS = 50
_EPS = 1e-08

_NC = 2
_NS = 16
_L = 16

_BINS_PAD = 64
_NCOPY = 8
_HWORDS = _BINS_PAD * _L * _NCOPY
_SC_CHUNK = 32768


def _pass1_body(nchunk, rblk, p_ref, s_ref, mm_ref):
    j = pl.program_id(0)
    vals = []
    for r in range(rblk):
        p0 = p_ref[:, 0, r, :]
        p1 = p_ref[:, 1, r, :]
        vals.append(p0 * p0 + p1 * p1)
    bmin = jnp.min(functools.reduce(jnp.minimum, vals))
    bmax = jnp.max(functools.reduce(jnp.maximum, vals))

    @pl.when(j == 0)
    def _():
        mm_ref[0] = bmin
        mm_ref[1] = bmax

    @pl.when(j > 0)
    def _():
        mm_ref[0] = jnp.minimum(mm_ref[0], bmin)
        mm_ref[1] = jnp.maximum(mm_ref[1], bmax)

    @pl.when(j == nchunk - 1)
    def _():
        s_ref[0, 0] = mm_ref[0]
        s_ref[0, 1] = mm_ref[1]


def _sc_minmax_body(rows, t_hbm, out, b0a, b1a, b0b, b1b, mmbuf, sem0, sem1):
    c = lax.axis_index("c")
    s = lax.axis_index("s")
    wid = c * _NS + s
    rchunk = 32
    nchunk = rows // 2 // rchunk
    r0 = c * (rows // 2)
    w = 512

    bufs = ((b0a, b1a), (b0b, b1b))
    sems = (sem0, sem1)
    copies = [None, None]

    def start(k, slot):
        ra = r0 + k * rchunk
        c0 = pltpu.async_copy(t_hbm.at[s, 0, pl.ds(ra, rchunk), :], bufs[slot][0], sems[slot])
        c1 = pltpu.async_copy(t_hbm.at[s, 1, pl.ds(ra, rchunk), :], bufs[slot][1], sems[slot])
        return (c0, c1)

    copies[0] = start(0, 0)
    mn = jnp.full((_L,), 3.4028235e38, jnp.float32)
    mx = jnp.zeros((_L,), jnp.float32)
    nvec = rchunk * w // _L
    for k in range(nchunk):
        cur = k % 2
        if k + 1 < nchunk:
            copies[1 - cur] = start(k + 1, 1 - cur)
        copies[cur][0].wait()
        copies[cur][1].wait()
        b0, b1 = bufs[cur]

        def body(i, carry):
            cmn, cmx = carry
            r = i // (w // _L)
            cc = (i % (w // _L)) * _L
            v0 = b0[r, pl.ds(cc, _L)]
            v1 = b1[r, pl.ds(cc, _L)]
            sq = v0 * v0 + v1 * v1
            return (jnp.minimum(cmn, sq), jnp.maximum(cmx, sq))

        mn, mx = lax.fori_loop(0, nvec, body, (mn, mx), unroll=8)

    mmbuf[pl.ds(0, _L)] = mn
    mmbuf[pl.ds(_L, _L)] = mx
    pltpu.sync_copy(mmbuf, out.at[wid])


def _binpass_body(rblk, p_ref, s_ref, mm_ref, o_ref, scr_ref):
    j = pl.program_id(0)

    @pl.when(j == 0)
    def _():
        mm = mm_ref[...]
        sq_mn = jnp.minimum(s_ref[0, 0], jnp.min(mm[:, 0:_L]))
        sq_mx = jnp.maximum(s_ref[0, 1], jnp.max(mm[:, _L : 2 * _L]))
        lo = jnp.maximum(jnp.sqrt(sq_mn), 1e-6)
        hi2 = jnp.maximum(jnp.sqrt(sq_mx), 1e-6)
        scr_ref[0] = lo
        scr_ref[1] = _BINS / jnp.maximum(hi2 - lo, 1e-12)

    mn = scr_ref[0]
    sc = scr_ref[1]
    w = p_ref.shape[-1]
    ci = lax.broadcasted_iota(jnp.int32, (16, w), 1)
    pat = (ci & (_L - 1)) + ((ci >> 4) & (_NCOPY - 1)) * (_BINS_PAD * _L)
    for r in range(rblk):
        p0 = p_ref[:, 0, r, :]
        p1 = p_ref[:, 1, r, :]
        v = jnp.maximum(jnp.sqrt(p0 * p0 + p1 * p1), 1e-6)
        t = (v - mn) * sc
        t = jnp.maximum(jnp.minimum(t, float(_BINS - 1)), 0.0)
        idx = t.astype(jnp.int32)
        o_ref[:, r * w : (r + 1) * w] = idx * _L + pat


def _sc_hist_body(n, fidx, out, buf0, buf1, hist, sem0, sem1):
    c = lax.axis_index("c")
    s = lax.axis_index("s")
    wid = c * _NS + s
    half = n // 2
    base = c * half

    zv = jnp.zeros((_L,), jnp.float32)

    @plsc.parallel_loop(0, _HWORDS // _L, unroll=4)
    def _zero(i):
        hist[pl.ds(i * _L, _L)] = zv

    ones = jnp.ones((_L,), jnp.float32)
    nchunk = half // _SC_CHUNK
    nvec = _SC_CHUNK // _L
    bufs = (buf0, buf1)
    sems = (sem0, sem1)
    copies = [None, None]
    copies[0] = pltpu.async_copy(fidx.at[s, pl.ds(base, _SC_CHUNK)], buf0, sem0)
    for k in range(nchunk):
        cur = k % 2
        if k + 1 < nchunk:
            copies[1 - cur] = pltpu.async_copy(
                fidx.at[s, pl.ds(base + (k + 1) * _SC_CHUNK, _SC_CHUNK)],
                bufs[1 - cur],
                sems[1 - cur],
            )
        copies[cur].wait()
        buf = bufs[cur]

        @plsc.parallel_loop(0, nvec, unroll=8)
        def body(i):
            fl = buf[pl.ds(i * _L, _L)]
            plsc.addupdate_scatter(hist, [fl], ones)

    pltpu.sync_copy(hist, out.at[wid])


def _fold_partials(xs):
    x = xs[0:16, :] + xs[16:32, :]
    acc = x[:, 0 : _BINS_PAD * _L]
    for j in range(1, _NCOPY):
        acc = acc + x[:, j * _BINS_PAD * _L : (j + 1) * _BINS_PAD * _L]
    return acc


def _pass2_body(hp_ref, ht1_ref, ht2_ref, o_ref):
    ht = _fold_partials(ht1_ref[...]) + _fold_partials(ht2_ref[...])
    x = jnp.concatenate([_fold_partials(hp_ref[...]), ht], axis=0)
    rows = lax.broadcasted_iota(jnp.int32, (_BINS_PAD * _L, 128), 0)
    cols = lax.broadcasted_iota(jnp.int32, (_BINS_PAD * _L, 128), 1)
    m = (rows // _L == cols).astype(jnp.float32)
    h = jnp.dot(x, m, preferred_element_type=jnp.float32)
    lanes = lax.broadcasted_iota(jnp.int32, (16, 128), 1)
    valid = lanes < _BINS
    hp = jnp.where(valid, h[0:16, :] + _EPS, 0.0)
    ht = jnp.where(valid, h[16:32, :] + _EPS, 0.0)
    hp = hp / jnp.sum(hp, axis=1, keepdims=True)
    ht = ht / jnp.sum(ht, axis=1, keepdims=True)
    kl = jnp.where(valid, ht * jnp.log(ht / hp), 0.0)
    o_ref[0, 0] = jnp.sum(kl) / 16.0


def kernel(y_pred, y_true):
    B, C, H, W = y_pred.shape
    N = H * W

    mesh = plsc.VectorSubcoreMesh(
        core_axis_name="c", subcore_axis_name="s", num_cores=_NC, num_subcores=_NS
    )

    sc_mm = functools.partial(
        pl.kernel,
        out_type=jax.ShapeDtypeStruct((2 * B, 2 * _L), jnp.float32),
        mesh=mesh,
        compiler_params=pltpu.CompilerParams(needs_layout_passes=False),
        scratch_types=[
            pltpu.VMEM((32, W), jnp.float32),
            pltpu.VMEM((32, W), jnp.float32),
            pltpu.VMEM((32, W), jnp.float32),
            pltpu.VMEM((32, W), jnp.float32),
            pltpu.VMEM((2 * _L,), jnp.float32),
            pltpu.SemaphoreType.DMA,
            pltpu.SemaphoreType.DMA,
        ],
    )(functools.partial(_sc_minmax_body, H))(y_true)

    ch1 = 16384
    rblk1 = ch1 // W
    stats = pl.pallas_call(
        functools.partial(_pass1_body, N // ch1, rblk1),
        grid=(N // ch1,),
        in_specs=[
            pl.BlockSpec((B, C, rblk1, W), lambda j: (0, 0, j, 0)),
        ],
        out_specs=pl.BlockSpec((1, 2), lambda j: (0, 0), memory_space=pltpu.SMEM),
        out_shape=jax.ShapeDtypeStruct((1, 2), jnp.float32),
        scratch_shapes=[pltpu.SMEM((2,), jnp.float32)],
    )(y_pred)

    ch = 8192
    rblk = ch // W
    nchunk = N // ch

    def binpass(arr, r0, nr):
        return pl.pallas_call(
            functools.partial(_binpass_body, rblk),
            grid=(nr,),
            in_specs=[
                pl.BlockSpec((B, C, rblk, W), lambda j: (0, 0, r0 + j, 0)),
                pl.BlockSpec((1, 2), lambda j: (0, 0), memory_space=pltpu.SMEM),
                pl.BlockSpec((2 * B, 2 * _L), lambda j: (0, 0)),
            ],
            out_specs=pl.BlockSpec((B, ch), lambda j: (0, j)),
            out_shape=jax.ShapeDtypeStruct((B, nr * ch), jnp.int32),
            scratch_shapes=[pltpu.SMEM((2,), jnp.float32)],
        )(arr, stats, sc_mm)

    def sc_hist(n):
        return functools.partial(
            pl.kernel,
            out_type=jax.ShapeDtypeStruct((2 * B, _HWORDS), jnp.float32),
            mesh=mesh,
            compiler_params=pltpu.CompilerParams(needs_layout_passes=False),
            scratch_types=[
                pltpu.VMEM((_SC_CHUNK,), jnp.int32),
                pltpu.VMEM((_SC_CHUNK,), jnp.int32),
                pltpu.VMEM((_HWORDS,), jnp.float32),
                pltpu.SemaphoreType.DMA,
                pltpu.SemaphoreType.DMA,
            ],
        )(functools.partial(_sc_hist_body, n))

    hists_p = sc_hist(N)(binpass(y_pred, 0, nchunk))
    hists_t1 = sc_hist(N // 2)(binpass(y_true, 0, nchunk // 2))
    hists_t2 = sc_hist(N // 2)(binpass(y_true, nchunk // 2, nchunk // 2))

    out = pl.pallas_call(
        _pass2_body,
        in_specs=[
            pl.BlockSpec((2 * B, _HWORDS), lambda: (0, 0)),
            pl.BlockSpec((2 * B, _HWORDS), lambda: (0, 0)),
            pl.BlockSpec((2 * B, _HWORDS), lambda: (0, 0)),
        ],
        out_specs=pl.BlockSpec(memory_space=pltpu.SMEM),
        out_shape=jax.ShapeDtypeStruct((1, 1), jnp.float32),
    )(hists_p, hists_t1, hists_t2)
    return out[0, 0]


# speedup vs baseline: 1.3985x; 1.1223x over previous
import functools

import jax
import jax.numpy as jnp
from jax import lax
from jax.experimental import pallas as pl
from jax.experimental.pallas import tpu as pltpu
from jax.experimental.pallas import tpu_sc as plsc

_BINS = 50
_EPS = 1e-08

_NC = 2
_NS = 16
_L = 16

_BINS_PAD = 64
_NCOPY = 8
_HWORDS = _BINS_PAD * _L * _NCOPY
_SC_CHUNK = 32768


def _pass1_body(nchunk, rblk, p_ref, s_ref, mm_ref):
    j = pl.program_id(0)
    vals = []
    for r in range(rblk):
        p0 = p_ref[:, 0, r, :]
        p1 = p_ref[:, 1, r, :]
        vals.append(p0 * p0 + p1 * p1)
    bmin = jnp.min(functools.reduce(jnp.minimum, vals))
    bmax = jnp.max(functools.reduce(jnp.maximum, vals))

    @pl.when(j == 0)
    def _():
        mm_ref[0] = bmin
        mm_ref[1] = bmax

    @pl.when(j > 0)
    def _():
        mm_ref[0] = jnp.minimum(mm_ref[0], bmin)
        mm_ref[1] = jnp.maximum(mm_ref[1], bmax)

    @pl.when(j == nchunk - 1)
    def _():
        s_ref[0, 0] = mm_ref[0]
        s_ref[0, 1] = mm_ref[1]


def _sc_minmax_body(rows, t_hbm, out, b0a, b1a, b0b, b1b, mmbuf, sem0, sem1):
    c = lax.axis_index("c")
    s = lax.axis_index("s")
    wid = c * _NS + s
    rchunk = 32
    nchunk = rows // 2 // rchunk
    r0 = c * (rows // 2)
    w = 512

    bufs = ((b0a, b1a), (b0b, b1b))
    sems = (sem0, sem1)
    copies = [None, None]

    def start(k, slot):
        ra = r0 + k * rchunk
        c0 = pltpu.async_copy(t_hbm.at[s, 0, pl.ds(ra, rchunk), :], bufs[slot][0], sems[slot])
        c1 = pltpu.async_copy(t_hbm.at[s, 1, pl.ds(ra, rchunk), :], bufs[slot][1], sems[slot])
        return (c0, c1)

    copies[0] = start(0, 0)
    mn = jnp.full((_L,), 3.4028235e38, jnp.float32)
    mx = jnp.zeros((_L,), jnp.float32)
    nvec = rchunk * w // _L
    for k in range(nchunk):
        cur = k % 2
        if k + 1 < nchunk:
            copies[1 - cur] = start(k + 1, 1 - cur)
        copies[cur][0].wait()
        copies[cur][1].wait()
        b0, b1 = bufs[cur]

        def body(i, carry):
            cmn, cmx = carry
            r = i // (w // _L)
            cc = (i % (w // _L)) * _L
            v0 = b0[r, pl.ds(cc, _L)]
            v1 = b1[r, pl.ds(cc, _L)]
            sq = v0 * v0 + v1 * v1
            return (jnp.minimum(cmn, sq), jnp.maximum(cmx, sq))

        mn, mx = lax.fori_loop(0, nvec, body, (mn, mx), unroll=8)

    mmbuf[pl.ds(0, _L)] = mn
    mmbuf[pl.ds(_L, _L)] = mx
    pltpu.sync_copy(mmbuf, out.at[wid])


def _binpass_body(rblk, p_ref, s_ref, mm_ref, o_ref, scr_ref):
    j = pl.program_id(0)

    @pl.when(j == 0)
    def _():
        mm = mm_ref[...]
        sq_mn = jnp.minimum(s_ref[0, 0], jnp.min(mm[:, 0:_L]))
        sq_mx = jnp.maximum(s_ref[0, 1], jnp.max(mm[:, _L : 2 * _L]))
        lo = jnp.maximum(jnp.sqrt(sq_mn), 1e-6)
        hi2 = jnp.maximum(jnp.sqrt(sq_mx), 1e-6)
        scr_ref[0] = lo
        scr_ref[1] = _BINS / jnp.maximum(hi2 - lo, 1e-12)

    mn = scr_ref[0]
    sc = scr_ref[1]
    w = p_ref.shape[-1]
    ci = lax.broadcasted_iota(jnp.int32, (16, w), 1)
    pat = (ci & (_L - 1)) + ((ci >> 4) & (_NCOPY - 1)) * (_BINS_PAD * _L)
    for r in range(rblk):
        p0 = p_ref[:, 0, r, :]
        p1 = p_ref[:, 1, r, :]
        v = jnp.maximum(jnp.sqrt(p0 * p0 + p1 * p1), 1e-6)
        t = (v - mn) * sc
        t = jnp.maximum(jnp.minimum(t, float(_BINS - 1)), 0.0)
        idx = t.astype(jnp.int32)
        o_ref[:, r * w : (r + 1) * w] = idx * _L + pat


def _sc_hist_body(n, fidx, out, buf0, buf1, hist, sem0, sem1):
    c = lax.axis_index("c")
    s = lax.axis_index("s")
    wid = c * _NS + s
    half = n // 2
    base = c * half

    zv = jnp.zeros((_L,), jnp.float32)

    @plsc.parallel_loop(0, _HWORDS // _L, unroll=4)
    def _zero(i):
        hist[pl.ds(i * _L, _L)] = zv

    ones = jnp.ones((_L,), jnp.float32)
    nchunk = half // _SC_CHUNK
    nvec = _SC_CHUNK // _L
    bufs = (buf0, buf1)
    sems = (sem0, sem1)
    copies = [None, None]
    copies[0] = pltpu.async_copy(fidx.at[s, pl.ds(base, _SC_CHUNK)], buf0, sem0)
    for k in range(nchunk):
        cur = k % 2
        if k + 1 < nchunk:
            copies[1 - cur] = pltpu.async_copy(
                fidx.at[s, pl.ds(base + (k + 1) * _SC_CHUNK, _SC_CHUNK)],
                bufs[1 - cur],
                sems[1 - cur],
            )
        copies[cur].wait()
        buf = bufs[cur]

        @plsc.parallel_loop(0, nvec, unroll=8)
        def body(i):
            fl = buf[pl.ds(i * _L, _L)]
            plsc.addupdate_scatter(hist, [fl], ones)

    pltpu.sync_copy(hist, out.at[wid])


def _fold_partials(xs):
    x = xs[0:16, :] + xs[16:32, :]
    acc = x[:, 0 : _BINS_PAD * _L]
    for j in range(1, _NCOPY):
        acc = acc + x[:, j * _BINS_PAD * _L : (j + 1) * _BINS_PAD * _L]
    return acc


def _pass2_body(hp_ref, ht1_ref, ht2_ref, o_ref):
    ht = _fold_partials(ht1_ref[...]) + _fold_partials(ht2_ref[...])
    x = jnp.concatenate([_fold_partials(hp_ref[...]), ht], axis=0)
    rows = lax.broadcasted_iota(jnp.int32, (_BINS_PAD * _L, 128), 0)
    cols = lax.broadcasted_iota(jnp.int32, (_BINS_PAD * _L, 128), 1)
    m = (rows // _L == cols).astype(jnp.float32)
    h = jnp.dot(x, m, preferred_element_type=jnp.float32)
    lanes = lax.broadcasted_iota(jnp.int32, (16, 128), 1)
    valid = lanes < _BINS
    hp = jnp.where(valid, h[0:16, :] + _EPS, 0.0)
    ht = jnp.where(valid, h[16:32, :] + _EPS, 0.0)
    hp = hp / jnp.sum(hp, axis=1, keepdims=True)
    ht = ht / jnp.sum(ht, axis=1, keepdims=True)
    kl = jnp.where(valid, ht * jnp.log(ht / hp), 0.0)
    o_ref[0, 0] = jnp.sum(kl) / 16.0


def kernel(y_pred, y_true):
    B, C, H, W = y_pred.shape
    N = H * W

    mesh = plsc.VectorSubcoreMesh(
        core_axis_name="c", subcore_axis_name="s", num_cores=_NC, num_subcores=_NS
    )

    sc_mm = functools.partial(
        pl.kernel,
        out_type=jax.ShapeDtypeStruct((2 * B, 2 * _L), jnp.float32),
        mesh=mesh,
        compiler_params=pltpu.CompilerParams(needs_layout_passes=False),
        scratch_types=[
            pltpu.VMEM((32, W), jnp.float32),
            pltpu.VMEM((32, W), jnp.float32),
            pltpu.VMEM((32, W), jnp.float32),
            pltpu.VMEM((32, W), jnp.float32),
            pltpu.VMEM((2 * _L,), jnp.float32),
            pltpu.SemaphoreType.DMA,
            pltpu.SemaphoreType.DMA,
        ],
    )(functools.partial(_sc_minmax_body, H))(y_true)

    ch1 = 16384
    rblk1 = ch1 // W
    stats = pl.pallas_call(
        functools.partial(_pass1_body, N // ch1, rblk1),
        grid=(N // ch1,),
        in_specs=[
            pl.BlockSpec((B, C, rblk1, W), lambda j: (0, 0, j, 0)),
        ],
        out_specs=pl.BlockSpec((1, 2), lambda j: (0, 0), memory_space=pltpu.SMEM),
        out_shape=jax.ShapeDtypeStruct((1, 2), jnp.float32),
        scratch_shapes=[pltpu.SMEM((2,), jnp.float32)],
    )(y_pred)

    ch = 16384
    rblk = ch // W
    nchunk = N // ch

    def binpass(arr, r0, nr):
        return pl.pallas_call(
            functools.partial(_binpass_body, rblk),
            grid=(nr,),
            in_specs=[
                pl.BlockSpec((B, C, rblk, W), lambda j: (0, 0, r0 + j, 0)),
                pl.BlockSpec((1, 2), lambda j: (0, 0), memory_space=pltpu.SMEM),
                pl.BlockSpec((2 * B, 2 * _L), lambda j: (0, 0)),
            ],
            out_specs=pl.BlockSpec((B, ch), lambda j: (0, j)),
            out_shape=jax.ShapeDtypeStruct((B, nr * ch), jnp.int32),
            scratch_shapes=[pltpu.SMEM((2,), jnp.float32)],
        )(arr, stats, sc_mm)

    def sc_hist(n):
        return functools.partial(
            pl.kernel,
            out_type=jax.ShapeDtypeStruct((2 * B, _HWORDS), jnp.float32),
            mesh=mesh,
            compiler_params=pltpu.CompilerParams(needs_layout_passes=False),
            scratch_types=[
                pltpu.VMEM((_SC_CHUNK,), jnp.int32),
                pltpu.VMEM((_SC_CHUNK,), jnp.int32),
                pltpu.VMEM((_HWORDS,), jnp.float32),
                pltpu.SemaphoreType.DMA,
                pltpu.SemaphoreType.DMA,
            ],
        )(functools.partial(_sc_hist_body, n))

    hists_p = sc_hist(N)(binpass(y_pred, 0, nchunk))
    hists_t1 = sc_hist(N // 2)(binpass(y_true, 0, nchunk // 2))
    hists_t2 = sc_hist(N // 2)(binpass(y_true, nchunk // 2, nchunk // 2))

    out = pl.pallas_call(
        _pass2_body,
        in_specs=[
            pl.BlockSpec((2 * B, _HWORDS), lambda: (0, 0)),
            pl.BlockSpec((2 * B, _HWORDS), lambda: (0, 0)),
            pl.BlockSpec((2 * B, _HWORDS), lambda: (0, 0)),
        ],
        out_specs=pl.BlockSpec(memory_space=pltpu.SMEM),
        out_shape=jax.ShapeDtypeStruct((1, 1), jnp.float32),
    )(hists_p, hists_t1, hists_t2)
    return out[0, 0]


# speedup vs baseline: 1.4327x; 1.0244x over previous
import functools

import jax
import jax.numpy as jnp
from jax import lax
from jax.experimental import pallas as pl
from jax.experimental.pallas import tpu as pltpu
from jax.experimental.pallas import tpu_sc as plsc

_BINS = 50
_EPS = 1e-08

_NC = 2
_NS = 16
_L = 16

_BINS_PAD = 64
_NCOPY = 8
_HWORDS = _BINS_PAD * _L * _NCOPY
_SC_CHUNK = 32768


def _pass1_body(nchunk, rblk, p_ref, s_ref, mm_ref):
    j = pl.program_id(0)
    vals = []
    for r in range(rblk):
        p0 = p_ref[:, 0, r, :]
        p1 = p_ref[:, 1, r, :]
        vals.append(p0 * p0 + p1 * p1)
    bmin = jnp.min(functools.reduce(jnp.minimum, vals))
    bmax = jnp.max(functools.reduce(jnp.maximum, vals))

    @pl.when(j == 0)
    def _():
        mm_ref[0] = bmin
        mm_ref[1] = bmax

    @pl.when(j > 0)
    def _():
        mm_ref[0] = jnp.minimum(mm_ref[0], bmin)
        mm_ref[1] = jnp.maximum(mm_ref[1], bmax)

    @pl.when(j == nchunk - 1)
    def _():
        s_ref[0, 0] = mm_ref[0]
        s_ref[0, 1] = mm_ref[1]


def _sc_minmax_body(rows, t_hbm, out, b0a, b1a, b0b, b1b, mmbuf, sem0, sem1):
    c = lax.axis_index("c")
    s = lax.axis_index("s")
    wid = c * _NS + s
    rchunk = 32
    nchunk = rows // 2 // rchunk
    r0 = c * (rows // 2)
    w = 512

    bufs = ((b0a, b1a), (b0b, b1b))
    sems = (sem0, sem1)
    copies = [None, None]

    def start(k, slot):
        ra = r0 + k * rchunk
        c0 = pltpu.async_copy(t_hbm.at[s, 0, pl.ds(ra, rchunk), :], bufs[slot][0], sems[slot])
        c1 = pltpu.async_copy(t_hbm.at[s, 1, pl.ds(ra, rchunk), :], bufs[slot][1], sems[slot])
        return (c0, c1)

    copies[0] = start(0, 0)
    mn = jnp.full((_L,), 3.4028235e38, jnp.float32)
    mx = jnp.zeros((_L,), jnp.float32)
    nvec = rchunk * w // _L
    for k in range(nchunk):
        cur = k % 2
        if k + 1 < nchunk:
            copies[1 - cur] = start(k + 1, 1 - cur)
        copies[cur][0].wait()
        copies[cur][1].wait()
        b0, b1 = bufs[cur]

        def body(i, carry):
            cmn, cmx = carry
            r = i // (w // _L)
            cc = (i % (w // _L)) * _L
            v0 = b0[r, pl.ds(cc, _L)]
            v1 = b1[r, pl.ds(cc, _L)]
            sq = v0 * v0 + v1 * v1
            return (jnp.minimum(cmn, sq), jnp.maximum(cmx, sq))

        mn, mx = lax.fori_loop(0, nvec, body, (mn, mx), unroll=8)

    mmbuf[pl.ds(0, _L)] = mn
    mmbuf[pl.ds(_L, _L)] = mx
    pltpu.sync_copy(mmbuf, out.at[wid])


def _binpass_body(rblk, p_ref, s_ref, mm_ref, o_ref, scr_ref):
    j = pl.program_id(0)

    @pl.when(j == 0)
    def _():
        mm = mm_ref[...]
        sq_mn = jnp.minimum(s_ref[0, 0], jnp.min(mm[:, 0:_L]))
        sq_mx = jnp.maximum(s_ref[0, 1], jnp.max(mm[:, _L : 2 * _L]))
        lo = jnp.maximum(jnp.sqrt(sq_mn), 1e-6)
        hi2 = jnp.maximum(jnp.sqrt(sq_mx), 1e-6)
        scr_ref[0] = lo
        scr_ref[1] = _BINS / jnp.maximum(hi2 - lo, 1e-12)

    mn = scr_ref[0]
    sc = scr_ref[1]
    w = p_ref.shape[-1]
    ci = lax.broadcasted_iota(jnp.int32, (16, w), 1)
    pat = (ci & (_L - 1)) + ((ci >> 4) & (_NCOPY - 1)) * (_BINS_PAD * _L)
    for r in range(rblk):
        p0 = p_ref[:, 0, r, :]
        p1 = p_ref[:, 1, r, :]
        v = jnp.maximum(jnp.sqrt(p0 * p0 + p1 * p1), 1e-6)
        t = (v - mn) * sc
        t = jnp.maximum(jnp.minimum(t, float(_BINS - 1)), 0.0)
        idx = t.astype(jnp.int32)
        o_ref[:, r * w : (r + 1) * w] = idx * _L + pat


def _sc_hist_body(n, fidx, out, buf0, buf1, hist, sem0, sem1):
    c = lax.axis_index("c")
    s = lax.axis_index("s")
    wid = c * _NS + s
    half = n // 2
    base = c * half

    zv = jnp.zeros((_L,), jnp.float32)

    @plsc.parallel_loop(0, _HWORDS // _L, unroll=4)
    def _zero(i):
        hist[pl.ds(i * _L, _L)] = zv

    ones = jnp.ones((_L,), jnp.float32)
    nchunk = half // _SC_CHUNK
    nvec = _SC_CHUNK // _L
    bufs = (buf0, buf1)
    sems = (sem0, sem1)
    copies = [None, None]
    copies[0] = pltpu.async_copy(fidx.at[s, pl.ds(base, _SC_CHUNK)], buf0, sem0)
    for k in range(nchunk):
        cur = k % 2
        if k + 1 < nchunk:
            copies[1 - cur] = pltpu.async_copy(
                fidx.at[s, pl.ds(base + (k + 1) * _SC_CHUNK, _SC_CHUNK)],
                bufs[1 - cur],
                sems[1 - cur],
            )
        copies[cur].wait()
        buf = bufs[cur]

        @plsc.parallel_loop(0, nvec, unroll=8)
        def body(i):
            fl = buf[pl.ds(i * _L, _L)]
            plsc.addupdate_scatter(hist, [fl], ones)

    pltpu.sync_copy(hist, out.at[wid])


def _fold_partials(xs):
    x = xs[0:16, :] + xs[16:32, :]
    acc = x[:, 0 : _BINS_PAD * _L]
    for j in range(1, _NCOPY):
        acc = acc + x[:, j * _BINS_PAD * _L : (j + 1) * _BINS_PAD * _L]
    return acc


def _pass2_body(hp_ref, ht1_ref, ht2_ref, o_ref):
    ht = _fold_partials(ht1_ref[...]) + _fold_partials(ht2_ref[...])
    x = jnp.concatenate([_fold_partials(hp_ref[...]), ht], axis=0)
    rows = lax.broadcasted_iota(jnp.int32, (_BINS_PAD * _L, 128), 0)
    cols = lax.broadcasted_iota(jnp.int32, (_BINS_PAD * _L, 128), 1)
    m = (rows // _L == cols).astype(jnp.float32)
    h = jnp.dot(x, m, preferred_element_type=jnp.float32)
    lanes = lax.broadcasted_iota(jnp.int32, (16, 128), 1)
    valid = lanes < _BINS
    hp = jnp.where(valid, h[0:16, :] + _EPS, 0.0)
    ht = jnp.where(valid, h[16:32, :] + _EPS, 0.0)
    hp = hp / jnp.sum(hp, axis=1, keepdims=True)
    ht = ht / jnp.sum(ht, axis=1, keepdims=True)
    kl = jnp.where(valid, ht * jnp.log(ht / hp), 0.0)
    o_ref[0, 0] = jnp.sum(kl) / 16.0


def kernel(y_pred, y_true):
    B, C, H, W = y_pred.shape
    N = H * W

    mesh = plsc.VectorSubcoreMesh(
        core_axis_name="c", subcore_axis_name="s", num_cores=_NC, num_subcores=_NS
    )

    sc_mm = functools.partial(
        pl.kernel,
        out_type=jax.ShapeDtypeStruct((2 * B, 2 * _L), jnp.float32),
        mesh=mesh,
        compiler_params=pltpu.CompilerParams(needs_layout_passes=False),
        scratch_types=[
            pltpu.VMEM((32, W), jnp.float32),
            pltpu.VMEM((32, W), jnp.float32),
            pltpu.VMEM((32, W), jnp.float32),
            pltpu.VMEM((32, W), jnp.float32),
            pltpu.VMEM((2 * _L,), jnp.float32),
            pltpu.SemaphoreType.DMA,
            pltpu.SemaphoreType.DMA,
        ],
    )(functools.partial(_sc_minmax_body, H))(y_true)

    ch1 = 16384
    rblk1 = ch1 // W
    stats = pl.pallas_call(
        functools.partial(_pass1_body, N // ch1, rblk1),
        grid=(N // ch1,),
        in_specs=[
            pl.BlockSpec((B, C, rblk1, W), lambda j: (0, 0, j, 0)),
        ],
        out_specs=pl.BlockSpec((1, 2), lambda j: (0, 0), memory_space=pltpu.SMEM),
        out_shape=jax.ShapeDtypeStruct((1, 2), jnp.float32),
        scratch_shapes=[pltpu.SMEM((2,), jnp.float32)],
    )(y_pred)

    ch = 32768
    rblk = ch // W
    nchunk = N // ch

    def binpass(arr, r0, nr):
        return pl.pallas_call(
            functools.partial(_binpass_body, rblk),
            grid=(nr,),
            in_specs=[
                pl.BlockSpec((B, C, rblk, W), lambda j: (0, 0, r0 + j, 0)),
                pl.BlockSpec((1, 2), lambda j: (0, 0), memory_space=pltpu.SMEM),
                pl.BlockSpec((2 * B, 2 * _L), lambda j: (0, 0)),
            ],
            out_specs=pl.BlockSpec((B, ch), lambda j: (0, j)),
            out_shape=jax.ShapeDtypeStruct((B, nr * ch), jnp.int32),
            scratch_shapes=[pltpu.SMEM((2,), jnp.float32)],
        )(arr, stats, sc_mm)

    def sc_hist(n):
        return functools.partial(
            pl.kernel,
            out_type=jax.ShapeDtypeStruct((2 * B, _HWORDS), jnp.float32),
            mesh=mesh,
            compiler_params=pltpu.CompilerParams(needs_layout_passes=False),
            scratch_types=[
                pltpu.VMEM((_SC_CHUNK,), jnp.int32),
                pltpu.VMEM((_SC_CHUNK,), jnp.int32),
                pltpu.VMEM((_HWORDS,), jnp.float32),
                pltpu.SemaphoreType.DMA,
                pltpu.SemaphoreType.DMA,
            ],
        )(functools.partial(_sc_hist_body, n))

    hists_p = sc_hist(N)(binpass(y_pred, 0, nchunk))
    hists_t1 = sc_hist(N // 2)(binpass(y_true, 0, nchunk // 2))
    hists_t2 = sc_hist(N // 2)(binpass(y_true, nchunk // 2, nchunk // 2))

    out = pl.pallas_call(
        _pass2_body,
        in_specs=[
            pl.BlockSpec((2 * B, _HWORDS), lambda: (0, 0)),
            pl.BlockSpec((2 * B, _HWORDS), lambda: (0, 0)),
            pl.BlockSpec((2 * B, _HWORDS), lambda: (0, 0)),
        ],
        out_specs=pl.BlockSpec(memory_space=pltpu.SMEM),
        out_shape=jax.ShapeDtypeStruct((1, 1), jnp.float32),
    )(hists_p, hists_t1, hists_t2)
    return out[0, 0]
